# baseline (speedup 1.0000x reference)
import functools

import jax
import jax.numpy as jnp
import numpy as np
from jax.experimental import pallas as pl

_B, _N, _M, _C2, _C1 = 8, 4096, 1024, 64, 64
_H = 128
_TN = 4096


def _fp_kernel(qs_ref, xyz2t_ref, kv_ref,
               w1_ref, b1_ref, w2_ref, b2_ref, out_ref):
    qs = qs_ref[0]
    q = qs[0:3, :]
    n1 = qs[3:6, :]
    r = xyz2t_ref[0]
    kv = kv_ref[0]

    q2 = q[0:1, :] ** 2 + q[1:2, :] ** 2 + q[2:3, :] ** 2
    d2 = ((r[:, 3:4] + q2)
          + (r[:, 0:1] * q[0:1, :]
             + r[:, 1:2] * q[1:2, :]
             + r[:, 2:3] * q[2:3, :]))

    d2w = d2
    dists, ndists, gfeats = [], [], []
    for k in range(3):
        mval = jnp.min(d2w, axis=0, keepdims=True)
        eq = d2w == mval
        oh = eq.astype(jnp.float32)
        g = jnp.dot(kv, oh, preferred_element_type=jnp.float32)
        nd2k = ((n1[0:1, :] - g[0:1, :]) ** 2
                + (n1[1:2, :] - g[1:2, :]) ** 2
                + (n1[2:3, :] - g[2:3, :]) ** 2)
        dists.append(jnp.sqrt(jnp.maximum(mval, 0.0)))
        ndists.append(jnp.sqrt(nd2k))
        gfeats.append(g[3:, :])
        if k < 2:
            d2w = jnp.where(eq, jnp.float32(np.inf), d2w)

    rd = [1.0 / jnp.maximum(d, 1e-10) for d in dists]
    rsum = rd[0] + rd[1] + rd[2]
    rn = [1.0 / jnp.maximum(d, 1e-10) for d in ndists]
    rnsum = rn[0] + rn[1] + rn[2]
    denom = 1.0 / (rsum * rnsum)
    w = [rd[k] * rn[k] * denom for k in range(3)]

    interp = w[0] * gfeats[0] + w[1] * gfeats[1] + w[2] * gfeats[2]
    feat = jnp.concatenate([interp, qs[6:, :]], axis=0)
    h = jnp.maximum(
        jnp.dot(w1_ref[:], feat, preferred_element_type=jnp.float32)
        + b1_ref[:], 0.0)
    out_ref[0] = jnp.maximum(
        jnp.dot(w2_ref[:], h, preferred_element_type=jnp.float32)
        + b2_ref[:], 0.0)


@functools.partial(jax.jit, static_argnames=("interpret",))
def kernel(xyz1, xyz2, norm1, norm2, points1, points2, W1, b1, W2, b2,
           interpret=False):
    qs = jnp.concatenate([xyz1, norm1, points1], axis=1)
    xyz2t = jnp.transpose(xyz2, (0, 2, 1))
    r2 = jnp.sum(xyz2t * xyz2t, axis=2, keepdims=True)
    xyz2t = jnp.concatenate([-2.0 * xyz2t, r2], axis=2)
    kv = jnp.concatenate([norm2, points2], axis=1)
    b1c = b1.reshape(_H, 1)
    b2c = b2.reshape(_H, 1)

    grid = (_B, _N // _TN)
    return pl.pallas_call(
        _fp_kernel,
        grid=grid,
        in_specs=[
            pl.BlockSpec((1, 6 + _C1, _TN), lambda b, i: (b, 0, i)),
            pl.BlockSpec((1, _M, 4), lambda b, i: (b, 0, 0)),
            pl.BlockSpec((1, 3 + _C2, _M), lambda b, i: (b, 0, 0)),
            pl.BlockSpec((_H, _C2 + _C1), lambda b, i: (0, 0)),
            pl.BlockSpec((_H, 1), lambda b, i: (0, 0)),
            pl.BlockSpec((_H, _H), lambda b, i: (0, 0)),
            pl.BlockSpec((_H, 1), lambda b, i: (0, 0)),
        ],
        out_specs=pl.BlockSpec((1, _H, _TN), lambda b, i: (b, 0, i)),
        out_shape=jax.ShapeDtypeStruct((_B, _H, _N), jnp.float32),
        interpret=interpret,
    )(qs, xyz2t, kv, W1, b1c, W2, b2c)


# speedup vs baseline: 1.0840x; 1.0840x over previous
import functools

import jax
import jax.numpy as jnp
import numpy as np
from jax.experimental import pallas as pl

_B, _N, _M, _C2, _C1 = 8, 4096, 1024, 64, 64
_H = 128
_TN = 4096


def _fp_kernel(xyz1_ref, xyz2t_ref, norm1_ref, kv_ref, p1_ref,
               w1_ref, b1_ref, w2_ref, b2_ref, out_ref):
    q = xyz1_ref[0]
    n1 = norm1_ref[0]
    r = xyz2t_ref[0]
    kv = kv_ref[0]

    q2 = q[0:1, :] ** 2 + q[1:2, :] ** 2 + q[2:3, :] ** 2
    d2 = ((r[:, 3:4] + q2)
          + (r[:, 0:1] * q[0:1, :]
             + r[:, 1:2] * q[1:2, :]
             + r[:, 2:3] * q[2:3, :]))

    d2w = d2
    dists, ndists, gfeats = [], [], []
    for k in range(3):
        mval = jnp.min(d2w, axis=0, keepdims=True)
        eq = d2w == mval
        oh = eq.astype(jnp.float32)
        g = jnp.dot(kv, oh, preferred_element_type=jnp.float32)
        nd2k = ((n1[0:1, :] - g[0:1, :]) ** 2
                + (n1[1:2, :] - g[1:2, :]) ** 2
                + (n1[2:3, :] - g[2:3, :]) ** 2)
        dists.append(jnp.sqrt(jnp.maximum(mval, 0.0)))
        ndists.append(jnp.sqrt(nd2k))
        gfeats.append(g[3:, :])
        if k < 2:
            d2w = jnp.where(eq, jnp.float32(np.inf), d2w)

    rd = [1.0 / jnp.maximum(d, 1e-10) for d in dists]
    rsum = rd[0] + rd[1] + rd[2]
    rn = [1.0 / jnp.maximum(d, 1e-10) for d in ndists]
    rnsum = rn[0] + rn[1] + rn[2]
    denom = 1.0 / (rsum * rnsum)
    w = [rd[k] * rn[k] * denom for k in range(3)]

    interp = w[0] * gfeats[0] + w[1] * gfeats[1] + w[2] * gfeats[2]
    feat = jnp.concatenate([interp, p1_ref[0]], axis=0)
    h = jnp.maximum(
        jnp.dot(w1_ref[:], feat, preferred_element_type=jnp.float32)
        + b1_ref[:], 0.0)
    out_ref[0] = jnp.maximum(
        jnp.dot(w2_ref[:], h, preferred_element_type=jnp.float32)
        + b2_ref[:], 0.0)


@functools.partial(jax.jit, static_argnames=("interpret",))
def kernel(xyz1, xyz2, norm1, norm2, points1, points2, W1, b1, W2, b2,
           interpret=False):
    xyz2t = jnp.transpose(xyz2, (0, 2, 1))
    r2 = jnp.sum(xyz2t * xyz2t, axis=2, keepdims=True)
    xyz2t = jnp.concatenate([-2.0 * xyz2t, r2], axis=2)
    kv = jnp.concatenate([norm2, points2], axis=1)
    b1c = b1.reshape(_H, 1)
    b2c = b2.reshape(_H, 1)

    grid = (_B, _N // _TN)
    return pl.pallas_call(
        _fp_kernel,
        grid=grid,
        in_specs=[
            pl.BlockSpec((1, 3, _TN), lambda b, i: (b, 0, i)),
            pl.BlockSpec((1, _M, 4), lambda b, i: (b, 0, 0)),
            pl.BlockSpec((1, 3, _TN), lambda b, i: (b, 0, i)),
            pl.BlockSpec((1, 3 + _C2, _M), lambda b, i: (b, 0, 0)),
            pl.BlockSpec((1, _C1, _TN), lambda b, i: (b, 0, i)),
            pl.BlockSpec((_H, _C2 + _C1), lambda b, i: (0, 0)),
            pl.BlockSpec((_H, 1), lambda b, i: (0, 0)),
            pl.BlockSpec((_H, _H), lambda b, i: (0, 0)),
            pl.BlockSpec((_H, 1), lambda b, i: (0, 0)),
        ],
        out_specs=pl.BlockSpec((1, _H, _TN), lambda b, i: (b, 0, i)),
        out_shape=jax.ShapeDtypeStruct((_B, _H, _N), jnp.float32),
        interpret=interpret,
    )(xyz1, xyz2t, norm1, kv, points1, W1, b1c, W2, b2c)


# speedup vs baseline: 1.0854x; 1.0013x over previous
import functools

import jax
import jax.numpy as jnp
import numpy as np
from jax.experimental import pallas as pl
from jax.experimental.pallas import tpu as pltpu

_B, _N, _M, _C2, _C1 = 8, 4096, 1024, 64, 64
_H = 128
_TN = 4096


def _fp_kernel(xyz1_ref, xyz2t_ref, norm1_ref, kv_ref, p1_ref,
               w1_ref, b1_ref, w2_ref, b2_ref, out_ref):
    q = xyz1_ref[0]
    n1 = norm1_ref[0]
    r = xyz2t_ref[0]
    kv = kv_ref[0]

    q2 = q[0:1, :] ** 2 + q[1:2, :] ** 2 + q[2:3, :] ** 2
    d2 = ((r[:, 3:4] + q2)
          + (r[:, 0:1] * q[0:1, :]
             + r[:, 1:2] * q[1:2, :]
             + r[:, 2:3] * q[2:3, :]))

    d2w = d2
    dists, ndists, gfeats = [], [], []
    for k in range(3):
        mval = jnp.min(d2w, axis=0, keepdims=True)
        eq = d2w == mval
        oh = eq.astype(jnp.float32)
        g = jnp.dot(kv, oh, preferred_element_type=jnp.float32)
        nd2k = ((n1[0:1, :] - g[0:1, :]) ** 2
                + (n1[1:2, :] - g[1:2, :]) ** 2
                + (n1[2:3, :] - g[2:3, :]) ** 2)
        dists.append(jnp.sqrt(jnp.maximum(mval, 0.0)))
        ndists.append(jnp.sqrt(nd2k))
        gfeats.append(g[3:, :])
        if k < 2:
            d2w = jnp.where(eq, jnp.float32(np.inf), d2w)

    rd = [1.0 / jnp.maximum(d, 1e-10) for d in dists]
    rsum = rd[0] + rd[1] + rd[2]
    rn = [1.0 / jnp.maximum(d, 1e-10) for d in ndists]
    rnsum = rn[0] + rn[1] + rn[2]
    denom = 1.0 / (rsum * rnsum)
    w = [rd[k] * rn[k] * denom for k in range(3)]

    interp = w[0] * gfeats[0] + w[1] * gfeats[1] + w[2] * gfeats[2]
    feat = jnp.concatenate([interp, p1_ref[0]], axis=0)
    h = jnp.maximum(
        jnp.dot(w1_ref[:], feat, preferred_element_type=jnp.float32)
        + b1_ref[:], 0.0)
    out_ref[0] = jnp.maximum(
        jnp.dot(w2_ref[:], h, preferred_element_type=jnp.float32)
        + b2_ref[:], 0.0)


@functools.partial(jax.jit, static_argnames=("interpret",))
def kernel(xyz1, xyz2, norm1, norm2, points1, points2, W1, b1, W2, b2,
           interpret=False):
    xyz2t = jnp.transpose(xyz2, (0, 2, 1))
    r2 = jnp.sum(xyz2t * xyz2t, axis=2, keepdims=True)
    xyz2t = jnp.concatenate([-2.0 * xyz2t, r2], axis=2)
    kv = jnp.concatenate([norm2, points2], axis=1)
    b1c = b1.reshape(_H, 1)
    b2c = b2.reshape(_H, 1)

    grid = (_B, _N // _TN)
    return pl.pallas_call(
        _fp_kernel,
        grid=grid,
        in_specs=[
            pl.BlockSpec((1, 3, _TN), lambda b, i: (b, 0, i)),
            pl.BlockSpec((1, _M, 4), lambda b, i: (b, 0, 0)),
            pl.BlockSpec((1, 3, _TN), lambda b, i: (b, 0, i)),
            pl.BlockSpec((1, 3 + _C2, _M), lambda b, i: (b, 0, 0)),
            pl.BlockSpec((1, _C1, _TN), lambda b, i: (b, 0, i)),
            pl.BlockSpec((_H, _C2 + _C1), lambda b, i: (0, 0)),
            pl.BlockSpec((_H, 1), lambda b, i: (0, 0)),
            pl.BlockSpec((_H, _H), lambda b, i: (0, 0)),
            pl.BlockSpec((_H, 1), lambda b, i: (0, 0)),
        ],
        out_specs=pl.BlockSpec((1, _H, _TN), lambda b, i: (b, 0, i)),
        out_shape=jax.ShapeDtypeStruct((_B, _H, _N), jnp.float32),
        compiler_params=pltpu.CompilerParams(
            dimension_semantics=("parallel", "parallel")),
        interpret=interpret,
    )(xyz1, xyz2t, norm1, kv, points1, W1, b1c, W2, b2c)


# speedup vs baseline: 1.0872x; 1.0016x over previous
import jax
import jax.numpy as jnp
import numpy as np
from jax.experimental import pallas as pl
from jax.experimental.pallas import tpu as pltpu

_B, _N, _M, _C2, _C1 = 8, 4096, 1024, 64, 64
_H = 128
_TN = 4096


def _fp_kernel(xyz1_ref, xyz2t_ref, norm1_ref, kv_ref, p1_ref,
               w1_ref, b1_ref, w2_ref, b2_ref, out_ref):
    q = xyz1_ref[0]
    n1 = norm1_ref[0]
    r = xyz2t_ref[0]
    kv = kv_ref[0]

    q2 = q[0:1, :] ** 2 + q[1:2, :] ** 2 + q[2:3, :] ** 2
    d2 = ((r[:, 3:4] + q2)
          + (r[:, 0:1] * q[0:1, :]
             + r[:, 1:2] * q[1:2, :]
             + r[:, 2:3] * q[2:3, :]))

    d2w = d2
    dists, ndists, gfeats = [], [], []
    for k in range(3):
        mval = jnp.min(d2w, axis=0, keepdims=True)
        eq = d2w == mval
        oh = eq.astype(jnp.float32)
        g = jnp.dot(kv, oh, preferred_element_type=jnp.float32)
        nd2k = ((n1[0:1, :] - g[0:1, :]) ** 2
                + (n1[1:2, :] - g[1:2, :]) ** 2
                + (n1[2:3, :] - g[2:3, :]) ** 2)
        dists.append(jnp.sqrt(jnp.maximum(mval, 0.0)))
        ndists.append(jnp.sqrt(nd2k))
        gfeats.append(g[3:, :])
        if k < 2:
            d2w = jnp.where(eq, jnp.float32(np.inf), d2w)

    rd = [1.0 / jnp.maximum(d, 1e-10) for d in dists]
    rsum = rd[0] + rd[1] + rd[2]
    rn = [1.0 / jnp.maximum(d, 1e-10) for d in ndists]
    rnsum = rn[0] + rn[1] + rn[2]
    denom = 1.0 / (rsum * rnsum)
    w = [rd[k] * rn[k] * denom for k in range(3)]

    interp = w[0] * gfeats[0] + w[1] * gfeats[1] + w[2] * gfeats[2]
    feat = jnp.concatenate([interp, p1_ref[0]], axis=0)
    h = jnp.maximum(
        jnp.dot(w1_ref[:], feat, preferred_element_type=jnp.float32)
        + b1_ref[:], 0.0)
    out_ref[0] = jnp.maximum(
        jnp.dot(w2_ref[:], h, preferred_element_type=jnp.float32)
        + b2_ref[:], 0.0)


@jax.jit
def kernel(xyz1, xyz2, norm1, norm2, points1, points2, W1, b1, W2, b2):
    xyz2t = jnp.transpose(xyz2, (0, 2, 1))
    r2 = jnp.sum(xyz2t * xyz2t, axis=2, keepdims=True)
    xyz2t = jnp.concatenate([-2.0 * xyz2t, r2], axis=2)
    kv = jnp.concatenate([norm2, points2], axis=1)
    b1c = b1.reshape(_H, 1)
    b2c = b2.reshape(_H, 1)

    grid = (_B, _N // _TN)
    return pl.pallas_call(
        _fp_kernel,
        grid=grid,
        in_specs=[
            pl.BlockSpec((1, 3, _TN), lambda b, i: (b, 0, i)),
            pl.BlockSpec((1, _M, 4), lambda b, i: (b, 0, 0)),
            pl.BlockSpec((1, 3, _TN), lambda b, i: (b, 0, i)),
            pl.BlockSpec((1, 3 + _C2, _M), lambda b, i: (b, 0, 0)),
            pl.BlockSpec((1, _C1, _TN), lambda b, i: (b, 0, i)),
            pl.BlockSpec((_H, _C2 + _C1), lambda b, i: (0, 0)),
            pl.BlockSpec((_H, 1), lambda b, i: (0, 0)),
            pl.BlockSpec((_H, _H), lambda b, i: (0, 0)),
            pl.BlockSpec((_H, 1), lambda b, i: (0, 0)),
        ],
        out_specs=pl.BlockSpec((1, _H, _TN), lambda b, i: (b, 0, i)),
        out_shape=jax.ShapeDtypeStruct((_B, _H, _N), jnp.float32),
        compiler_params=pltpu.CompilerParams(
            dimension_semantics=("parallel", "parallel")),
    )(xyz1, xyz2t, norm1, kv, points1, W1, b1c, W2, b2c)


# speedup vs baseline: 1.1407x; 1.0493x over previous
import jax
import jax.numpy as jnp
import numpy as np
from jax.experimental import pallas as pl
from jax.experimental.pallas import tpu as pltpu

_B, _N, _M, _C2, _C1 = 8, 4096, 1024, 64, 64
_H = 128
_TN = 4096


def _fp_kernel(xyz1_ref, xyz2t_ref, norm1_ref, kv_ref, p1_ref,
               w1_ref, b1_ref, w2_ref, b2_ref, out_ref):
    q = xyz1_ref[0]
    n1 = norm1_ref[0]
    r = xyz2t_ref[0]
    kv = kv_ref[0]

    q2 = q[0:1, :] ** 2 + q[1:2, :] ** 2 + q[2:3, :] ** 2
    d2 = ((r[:, 3:4] + q2)
          + (r[:, 0:1] * q[0:1, :]
             + r[:, 1:2] * q[1:2, :]
             + r[:, 2:3] * q[2:3, :]))

    d2w = d2
    dists, ndists, gfeats = [], [], []
    for k in range(3):
        mval = jnp.min(d2w, axis=0, keepdims=True)
        eq = d2w == mval
        oh = eq.astype(jnp.float32)
        g = jnp.dot(kv, oh, preferred_element_type=jnp.float32)
        nd2k = ((n1[0:1, :] - g[0:1, :]) ** 2
                + (n1[1:2, :] - g[1:2, :]) ** 2
                + (n1[2:3, :] - g[2:3, :]) ** 2)
        dists.append(jnp.sqrt(jnp.maximum(mval, 0.0)))
        ndists.append(jnp.sqrt(nd2k))
        gfeats.append(g[3:, :])
        if k < 2:
            d2w = jnp.where(eq, jnp.float32(np.inf), d2w)

    rd = [1.0 / jnp.maximum(d, 1e-10) for d in dists]
    rsum = rd[0] + rd[1] + rd[2]
    rn = [1.0 / jnp.maximum(d, 1e-10) for d in ndists]
    rnsum = rn[0] + rn[1] + rn[2]
    denom = 1.0 / (rsum * rnsum)
    w = [rd[k] * rn[k] * denom for k in range(3)]

    interp = w[0] * gfeats[0] + w[1] * gfeats[1] + w[2] * gfeats[2]
    feat = jnp.concatenate([interp, p1_ref[0]], axis=0)
    h = jnp.maximum(
        jnp.dot(w1_ref[:], feat, preferred_element_type=jnp.float32)
        + b1_ref[:], 0.0)
    out_ref[0] = jnp.maximum(
        jnp.dot(w2_ref[:], h, preferred_element_type=jnp.float32)
        + b2_ref[:], 0.0)


@jax.jit
def kernel(xyz1, xyz2, norm1, norm2, points1, points2, W1, b1, W2, b2):
    xyz2t = jnp.transpose(xyz2, (0, 2, 1))
    r2 = jnp.sum(xyz2t * xyz2t, axis=2, keepdims=True)
    xyz2t = jnp.concatenate([-2.0 * xyz2t, r2], axis=2)
    kv = jnp.concatenate([norm2, points2], axis=1)
    b1c = b1.reshape(_H, 1)
    b2c = b2.reshape(_H, 1)

    grid = (_B, _N // _TN)
    return pl.pallas_call(
        _fp_kernel,
        grid=grid,
        in_specs=[
            pl.BlockSpec((1, 3, _TN), lambda b, i: (b, 0, i)),
            pl.BlockSpec((1, _M, 4), lambda b, i: (b, 0, 0)),
            pl.BlockSpec((1, 3, _TN), lambda b, i: (b, 0, i)),
            pl.BlockSpec((1, 3 + _C2, _M), lambda b, i: (b, 0, 0)),
            pl.BlockSpec((1, _C1, _TN), lambda b, i: (b, 0, i)),
            pl.BlockSpec((_H, _C2 + _C1), lambda b, i: (0, 0)),
            pl.BlockSpec((_H, 1), lambda b, i: (0, 0)),
            pl.BlockSpec((_H, _H), lambda b, i: (0, 0)),
            pl.BlockSpec((_H, 1), lambda b, i: (0, 0)),
        ],
        out_specs=pl.BlockSpec((1, _H, _TN), lambda b, i: (b, 0, i)),
        out_shape=jax.ShapeDtypeStruct((_B, _H, _N), jnp.float32),
        compiler_params=pltpu.CompilerParams(
            dimension_semantics=("parallel", "parallel"),
            allow_input_fusion=[True] * 9),
    )(xyz1, xyz2t, norm1, kv, points1, W1, b1c, W2, b2c)
